# baseline (speedup 1.0000x reference)
import jax
import jax.numpy as jnp
from jax import lax
from jax.experimental import pallas as pl
from jax.experimental.pallas import tpu as pltpu
from jax.experimental.pallas import tpu_sc as plsc

_L = 16
_CW = 512
_GRP = 16
_BG = 8


def _sc_scan_gather(sid, tableT, batch, dim, nc, ns):
  nw = nc * ns
  v_users = tableT.shape[1]
  n_main = v_users // _CW
  tail_base = n_main * _CW
  tail_w = v_users - tail_base
  s_per_w = batch // nw
  n_blocks = s_per_w // (_GRP * _BG)
  mesh = plsc.VectorSubcoreMesh(core_axis_name="c", subcore_axis_name="s")

  def stateful(refs):
    sid_ref, tab_ref, tailsrc_ref, g_ref = refs

    bpg = _GRP * _BG

    @pl.core_map(
        mesh,
        compiler_params=pltpu.CompilerParams(needs_layout_passes=False),
        scratch_shapes=[
            pltpu.VMEM((s_per_w + _L,), jnp.int32),
            pltpu.VMEM((dim, _CW), jnp.float32),
            pltpu.VMEM((dim, max(tail_w, 8)), jnp.float32),
            pltpu.VMEM((1, dim, 128), jnp.float32),
            pltpu.VMEM(((bpg + 2) * _L,), jnp.int32),
            pltpu.VMEM(((bpg + 2) * _L,), jnp.int32),
            pltpu.SemaphoreType.DMA,
        ],
    )
    def _(myids_v, chunk_v, tail_v, cb_v, winb_v, winc_v, sem):
      cc = lax.axis_index("c")
      t = lax.axis_index("s")
      w = t * nc + cc
      base_slot = w * s_per_w
      zi = jnp.full((_L,), 0, jnp.int32)

      pltpu.sync_copy(sid_ref.at[pl.ds(base_slot, s_per_w)],
                      myids_v.at[pl.ds(0, s_per_w)])
      if tail_w:
        pltpu.sync_copy(tailsrc_ref, tail_v.at[:, pl.ds(0, tail_w)])

      def do_block(blk, carry0):
        bbase = blk * bpg

        def body1(c, st):
          nw, cur, cstart = st
          u = myids_v[pl.ds(bbase + c, _L)][0]
          wb = jnp.minimum(lax.shift_right_logical(u, 9),
                           jnp.int32(n_main - 1)) * _CW
          is_new = (wb != cur).astype(jnp.int32)
          nw2 = nw + is_new
          cs2 = jnp.where(is_new == 1, c, cstart)
          slot = nw2 - 1
          winb_v[pl.ds(slot * _L, _L)] = zi + wb
          winc_v[pl.ds(slot * _L, _L)] = zi + cs2
          return nw2, wb, cs2

        nw, _, _ = lax.fori_loop(
            0, bpg, body1,
            (jnp.int32(0), jnp.int32(-1), jnp.int32(0)))
        winc_v[pl.ds(nw * _L, _L)] = zi + jnp.int32(bpg)

        def body2(k0, carry):
          k = jnp.minimum(k0, nw - 1)
          wb = winb_v[pl.ds(k * _L, _L)][0]
          cs = winc_v[pl.ds(k * _L, _L)][0]
          cn = winc_v[pl.ds(k * _L + _L, _L)][0]
          pltpu.async_copy(
              tab_ref.at[:, pl.ds(pl.multiple_of(wb, 128), _CW)],
              chunk_v, sem)
          pltpu.make_async_copy(
              tab_ref.at[:, pl.ds(0, _CW)], chunk_v, sem).wait()
          gs = lax.shift_right_logical(cs, 4)
          ge = lax.shift_right_logical(cn - 1, 4)
          for g in range(_BG):
            if True:
              vec = myids_v[pl.ds(bbase + g * _GRP, _GRP)]
              m_main = jnp.logical_and(vec >= wb, vec < wb + _CW)
              ul_m = jnp.maximum(jnp.minimum(vec - wb, _CW - 1), 0)
              if tail_w:
                m_tail = vec >= tail_base
                ul_t = jnp.maximum(
                    jnp.minimum(vec - tail_base, tail_w - 1), 0)
              for q in range(dim):
                vm = plsc.load_gather(chunk_v, [zi + q, ul_m])
                old = cb_v[0, q, pl.ds(g * _GRP, _GRP)]
                new = jnp.where(m_main, vm, old)
                if tail_w:
                  vt = plsc.load_gather(tail_v, [zi + q, ul_t])
                  new = jnp.where(m_tail, vt, new)
                cb_v[0, q, pl.ds(g * _GRP, _GRP)] = new
          return carry

        lax.fori_loop(0, nw + 1, body2, jnp.int32(0))

        pl.delay(64)
        gb = w * n_blocks + blk
        pltpu.sync_copy(cb_v, g_ref.at[pl.ds(gb, 1)])
        return carry0

      lax.fori_loop(0, n_blocks, do_block, jnp.int32(0))

  g0 = jnp.zeros((batch // 128, dim, 128), jnp.float32)
  tailsrc = tableT[:, tail_base:] if tail_w else tableT[:, :8]
  g = pl.run_state(stateful)((sid, tableT, tailsrc, g0))[-1]
  return g


def _mlp_body(g_ref, w1_ref, b1_ref, w2_ref, b2_ref, o_ref):
  nb = g_ref.shape[0]
  for k in range(nb):
    h = lax.dot_general(g_ref[k], w1_ref[...], (((0,), (0,)), ((), ())),
                        preferred_element_type=jnp.float32)
    h = jnp.maximum(h + b1_ref[...], 0.0)
    o = jnp.dot(h, w2_ref[...], preferred_element_type=jnp.float32)
    o_ref[pl.ds(k * 128, 128), :] = o + b2_ref[...]


def _make_tc_mlp(batch, dim, hidden, blk):
  nb = blk // 128
  return pl.pallas_call(
      _mlp_body,
      grid=(batch // blk,),
      in_specs=[
          pl.BlockSpec((nb, dim, 128), lambda i: (i, 0, 0)),
          pl.BlockSpec((dim, hidden), lambda i: (0, 0)),
          pl.BlockSpec((1, hidden), lambda i: (0, 0)),
          pl.BlockSpec((hidden, dim), lambda i: (0, 0)),
          pl.BlockSpec((1, dim), lambda i: (0, 0)),
      ],
      out_specs=pl.BlockSpec((blk, dim), lambda i: (i, 0)),
      out_shape=jax.ShapeDtypeStruct((batch, dim), jnp.float32),
  )


def _sc_unpermute(spos, rows_sorted, batch, dim, nc, ns):
  nw = nc * ns
  s_per_w = batch // nw
  mesh = plsc.VectorSubcoreMesh(core_axis_name="c", subcore_axis_name="s")

  def stateful(refs):
    pos_ref, rows_ref, out_ref = refs

    @pl.core_map(
        mesh,
        scratch_shapes=[
            pltpu.VMEM((s_per_w,), jnp.int32),
            pltpu.VMEM((s_per_w, dim), jnp.float32),
            pltpu.SemaphoreType.DMA,
        ],
    )
    def _(pos_v, rows_v, sem):
      cc = lax.axis_index("c")
      t = lax.axis_index("s")
      w = t * nc + cc
      base = w * s_per_w
      pltpu.sync_copy(pos_ref.at[pl.ds(base, s_per_w)], pos_v)
      pltpu.sync_copy(rows_ref.at[pl.ds(base, s_per_w)], rows_v)

      def fire(g, carry):
        vec = pos_v[pl.ds(g * _L, _L)]
        for j in range(_L):
          pos = vec[j]
          pltpu.async_copy(
              rows_v.at[pl.ds(g * _L + j, 1)],
              out_ref.at[pl.ds(pos, 1)], sem)
        return carry

      lax.fori_loop(0, s_per_w // _L, fire, 0)
      pltpu.make_async_copy(
          rows_v, out_ref.at[pl.ds(0, s_per_w)], sem).wait()

  out0 = jnp.zeros((batch, dim), jnp.float32)
  return pl.run_state(stateful)((spos, rows_sorted, out0))[2]


@jax.jit
def kernel(user_ids, emb_table, W1, b1, W2, b2):
  batch = user_ids.shape[0]
  num_users, dim = emb_table.shape
  hidden = W1.shape[1]

  info = plsc.get_sparse_core_info()
  nc, ns = info.num_cores, info.num_subcores

  ids32 = user_ids.astype(jnp.int32)
  tableT = emb_table.T
  sid, spos = lax.sort_key_val(ids32, lax.iota(jnp.int32, batch))

  g = _sc_scan_gather(sid, tableT, batch, dim, nc, ns)
  mlp = _make_tc_mlp(batch, dim, hidden, blk=2048)
  rows_sorted = mlp(g, W1, b1.reshape(1, hidden), W2, b2.reshape(1, dim))
  return _sc_unpermute(spos, rows_sorted, batch, dim, nc, ns)


# speedup vs baseline: 1.7973x; 1.7973x over previous
import functools

import jax
import jax.numpy as jnp
from jax import lax
from jax.experimental import pallas as pl
from jax.experimental.pallas import tpu as pltpu
from jax.experimental.pallas import tpu_sc as plsc


def _sc_gather(ids32, table, batch, dim, nw, nc):
  b_per_w = batch // nw
  mesh = plsc.VectorSubcoreMesh(core_axis_name="c", subcore_axis_name="s")

  def stateful(refs):
    idx_ref, table_ref, out_ref = refs

    @pl.core_map(
        mesh,
        scratch_shapes=[
            pltpu.VMEM((b_per_w,), jnp.int32),
            pltpu.VMEM((b_per_w, dim), jnp.float32),
            pltpu.SemaphoreType.DMA,
        ],
    )
    def _(idx_v, rows_v, sem):
      wid = lax.axis_index("s") * nc + lax.axis_index("c")
      base = wid * b_per_w
      pltpu.sync_copy(idx_ref.at[pl.ds(base, b_per_w)], idx_v)

      def fire(g, carry):
        vec = idx_v[pl.ds(g * 16, 16)]
        for j in range(16):
          row = vec[j]
          pltpu.async_copy(
              table_ref.at[pl.ds(row, 1)],
              rows_v.at[pl.ds(g * 16 + j, 1)], sem)
        return carry

      lax.fori_loop(0, b_per_w // 16, fire, 0)
      pltpu.make_async_copy(
          table_ref.at[pl.ds(0, b_per_w)], rows_v, sem).wait()
      pltpu.sync_copy(rows_v, out_ref.at[pl.ds(base, b_per_w)])

  out_init = jnp.zeros((batch, dim), jnp.float32)
  _, _, out = pl.run_state(stateful)((ids32, table, out_init))
  return out


def _mlp_body(x_ref, w1_ref, b1_ref, w2_ref, b2_ref, o_ref):
  h = jnp.dot(x_ref[...], w1_ref[...], preferred_element_type=jnp.float32)
  h = jnp.maximum(h + b1_ref[...], 0.0)
  o = jnp.dot(h, w2_ref[...], preferred_element_type=jnp.float32)
  o_ref[...] = o + b2_ref[...]


def _make_tc_mlp(batch, dim, hidden, blk):
  grid = batch // blk
  return pl.pallas_call(
      _mlp_body,
      grid=(grid,),
      in_specs=[
          pl.BlockSpec((blk, dim), lambda i: (i, 0)),
          pl.BlockSpec((dim, hidden), lambda i: (0, 0)),
          pl.BlockSpec((1, hidden), lambda i: (0, 0)),
          pl.BlockSpec((hidden, dim), lambda i: (0, 0)),
          pl.BlockSpec((1, dim), lambda i: (0, 0)),
      ],
      out_specs=pl.BlockSpec((blk, dim), lambda i: (i, 0)),
      out_shape=jax.ShapeDtypeStruct((batch, dim), jnp.float32),
  )


@jax.jit
def kernel(user_ids, emb_table, W1, b1, W2, b2):
  batch = user_ids.shape[0]
  num_users, dim = emb_table.shape
  hidden = W1.shape[1]

  info = plsc.get_sparse_core_info()
  nw = info.num_cores * info.num_subcores

  ids32 = user_ids.astype(jnp.int32)
  gathered = _sc_gather(ids32, emb_table, batch, dim, nw, info.num_cores)

  mlp = _make_tc_mlp(batch, dim, hidden, blk=2048)
  return mlp(gathered, W1, b1.reshape(1, hidden), W2, b2.reshape(1, dim))
